# Initial kernel scaffold; baseline (speedup 1.0000x reference)
import jax
import jax.numpy as jnp
from jax.experimental import pallas as pl


def kernel(x, w1, b1, w2, b2):
    raise NotImplementedError("write your pallas kernel here")


# speedup vs baseline: 1.5163x; 1.5163x over previous
import functools

import jax
import jax.numpy as jnp
from jax.experimental import pallas as pl
from jax.experimental.pallas import tpu as pltpu

_LANES = 128


def _round_up(v, m):
    return ((v + m - 1) // m) * m


def _mlp_exp_kernel(x_ref, w1_ref, b1_ref, w2_ref, b2_ref, fr_ref, x0_ref):
    h = x0_ref.shape[-1]
    x = x_ref[...]
    x0 = jnp.dot(x, w1_ref[...],
                 preferred_element_type=jnp.float32) + b1_ref[...]
    x0_ref[...] = x0[:, :h]
    z = jnp.dot(x0, w2_ref[...],
                preferred_element_type=jnp.float32) + b2_ref[...]
    fr_ref[...] = jnp.exp(z)


@functools.partial(jax.jit, static_argnames=("block_b",))
def _lnp_forward(x, w1, b1, w2, b2, *, block_b=2048):
    B, D = x.shape
    H = w1.shape[0]
    N = w2.shape[0]

    Hp = _round_up(H, _LANES)
    Np = _round_up(N, _LANES)

    TB = min(block_b, _round_up(B, 8))
    Bp = _round_up(B, TB)
    x_in = x
    if Bp != B:
        x_in = jnp.zeros((Bp, D), x.dtype).at[:B, :].set(x)

    w1_p = jnp.zeros((D, Hp), jnp.float32).at[:, :H].set(w1.T)
    b1_p = jnp.zeros((1, Hp), jnp.float32).at[0, :H].set(b1)
    w2_p = jnp.zeros((Hp, Np), jnp.float32).at[:H, :N].set(w2.T)
    b2_p = jnp.zeros((1, Np), jnp.float32).at[0, :N].set(b2)

    fr_p, x0 = pl.pallas_call(
        _mlp_exp_kernel,
        out_shape=(
            jax.ShapeDtypeStruct((Bp, Np), jnp.float32),
            jax.ShapeDtypeStruct((Bp, H), jnp.float32),
        ),
        grid=(Bp // TB,),
        in_specs=[
            pl.BlockSpec((TB, D), lambda i: (i, 0)),
            pl.BlockSpec((D, Hp), lambda i: (0, 0)),
            pl.BlockSpec((1, Hp), lambda i: (0, 0)),
            pl.BlockSpec((Hp, Np), lambda i: (0, 0)),
            pl.BlockSpec((1, Np), lambda i: (0, 0)),
        ],
        out_specs=(
            pl.BlockSpec((TB, Np), lambda i: (i, 0)),
            pl.BlockSpec((TB, H), lambda i: (i, 0)),
        ),
        compiler_params=pltpu.CompilerParams(
            dimension_semantics=("parallel",),
        ),
    )(x_in, w1_p, b1_p, w2_p, b2_p)

    if Bp != B or Np != N:
        return fr_p[:B, :N], x0[:B, :]
    return fr_p, x0


def kernel(x, w1, b1, w2, b2):
    return _lnp_forward(x, w1, b1, w2, b2, block_b=2048)
